# Initial kernel scaffold; baseline (speedup 1.0000x reference)
import jax
import jax.numpy as jnp
from jax.experimental import pallas as pl


def kernel(hidden_states, gate_w, Wg, Wu, Wd, sWg, sWu, sWd, shared_gate_w):
    raise NotImplementedError("write your pallas kernel here")


# speedup vs baseline: 1.1467x; 1.1467x over previous
import functools

import jax
import jax.numpy as jnp
from jax.experimental import pallas as pl
from jax.experimental.pallas import tpu as pltpu

B, S, D = 1, 2048, 2048
E, TOP_K = 8, 2
DFF = 1408
DSH = 5632
T = B * S

BT = 512
BTS = 512
BF = 512
NF = DSH // BF


def _router_body(x_ref, gw_ref, logits_ref, dw_ref):
    x = x_ref[...].astype(jnp.bfloat16)
    logits = jax.lax.dot_general(
        x, gw_ref[...].astype(jnp.bfloat16), (((1,), (0,)), ((), ())),
        preferred_element_type=jnp.float32)
    logits_ref[...] = logits
    m = jnp.max(logits, axis=-1, keepdims=True)
    ex = jnp.exp(logits - m)
    p = ex / jnp.sum(ex, axis=-1, keepdims=True)
    lane = jax.lax.broadcasted_iota(jnp.int32, p.shape, 1)
    m1 = jnp.max(p, axis=-1, keepdims=True)
    i1 = jnp.min(jnp.where(p >= m1, lane, E), axis=-1, keepdims=True)
    oh1 = lane == i1
    p2 = jnp.where(oh1, -1.0, p)
    m2 = jnp.max(p2, axis=-1, keepdims=True)
    i2 = jnp.min(jnp.where(p2 >= m2, lane, E), axis=-1, keepdims=True)
    oh2 = lane == i2
    denom = m1 + m2
    dw_ref[...] = (jnp.where(oh1, m1, 0.0) + jnp.where(oh2, m2, 0.0)) / denom


def _moe_body(x_ref, dw_ref, wg_ref, wu_ref, wd_ref, out_ref):
    e = pl.program_id(1)
    x = x_ref[...]
    g = jnp.dot(x, wg_ref[0], preferred_element_type=jnp.float32)
    u = jnp.dot(x, wu_ref[0], preferred_element_type=jnp.float32)
    h = (g * jax.nn.sigmoid(g) * u).astype(jnp.bfloat16)
    y = jnp.dot(h, wd_ref[0], preferred_element_type=jnp.float32)
    lane = jax.lax.broadcasted_iota(jnp.int32, dw_ref.shape, 1)
    scale = jnp.sum(jnp.where(lane == e, dw_ref[...], 0.0), axis=-1,
                    keepdims=True)
    contrib = scale * y

    @pl.when(e == 0)
    def _():
        out_ref[...] = contrib

    @pl.when(e > 0)
    def _():
        out_ref[...] = out_ref[...] + contrib


def _shared_body(x_ref, wg_ref, wu_ref, wd_ref, sgw_ref, moe_ref, out_ref):
    f = pl.program_id(1)
    x = x_ref[...]
    g = jnp.dot(x, wg_ref[...], preferred_element_type=jnp.float32)
    u = jnp.dot(x, wu_ref[...], preferred_element_type=jnp.float32)
    h = (g * jax.nn.sigmoid(g) * u).astype(jnp.bfloat16)
    partial = jnp.dot(h, wd_ref[...], preferred_element_type=jnp.float32)
    prev = jnp.where(f == 0, jnp.zeros_like(partial), out_ref[...])
    acc = prev + partial

    @pl.when(f < NF - 1)
    def _():
        out_ref[...] = acc

    @pl.when(f == NF - 1)
    def _():
        xf = x.astype(jnp.float32)
        gate_logit = jnp.sum(xf * sgw_ref[...], axis=-1, keepdims=True)
        gate = jax.nn.sigmoid(gate_logit)
        out_ref[...] = acc * gate + moe_ref[...]


@jax.jit
def kernel(hidden_states, gate_w, Wg, Wu, Wd, sWg, sWu, sWd, shared_gate_w):
    x = hidden_states.reshape(T, D)
    logits, dw = pl.pallas_call(
        _router_body,
        out_shape=(
            jax.ShapeDtypeStruct((T, E), jnp.float32),
            jax.ShapeDtypeStruct((T, E), jnp.float32),
        ),
    )(x, gate_w)

    x_bf = x.astype(jnp.bfloat16)
    Wg_bf = Wg.astype(jnp.bfloat16)
    Wu_bf = Wu.astype(jnp.bfloat16)
    Wd_bf = Wd.astype(jnp.bfloat16)

    moe_out = pl.pallas_call(
        _moe_body,
        grid=(T // BT, E),
        in_specs=[
            pl.BlockSpec((BT, D), lambda i, e: (i, 0)),
            pl.BlockSpec((BT, E), lambda i, e: (i, 0)),
            pl.BlockSpec((1, D, DFF), lambda i, e: (e, 0, 0)),
            pl.BlockSpec((1, D, DFF), lambda i, e: (e, 0, 0)),
            pl.BlockSpec((1, DFF, D), lambda i, e: (e, 0, 0)),
        ],
        out_specs=pl.BlockSpec((BT, D), lambda i, e: (i, 0)),
        out_shape=jax.ShapeDtypeStruct((T, D), jnp.float32),
        compiler_params=pltpu.CompilerParams(
            dimension_semantics=("parallel", "arbitrary")),
    )(x_bf, dw, Wg_bf, Wu_bf, Wd_bf)

    sWg_bf = sWg.astype(jnp.bfloat16)
    sWu_bf = sWu.astype(jnp.bfloat16)
    sWd_bf = sWd.astype(jnp.bfloat16)
    sgw_t = shared_gate_w.reshape(1, D)

    final = pl.pallas_call(
        _shared_body,
        grid=(T // BTS, NF),
        in_specs=[
            pl.BlockSpec((BTS, D), lambda i, f: (i, 0)),
            pl.BlockSpec((D, BF), lambda i, f: (0, f)),
            pl.BlockSpec((D, BF), lambda i, f: (0, f)),
            pl.BlockSpec((BF, D), lambda i, f: (f, 0)),
            pl.BlockSpec((1, D), lambda i, f: (0, 0)),
            pl.BlockSpec((BTS, D), lambda i, f: (i, 0)),
        ],
        out_specs=pl.BlockSpec((BTS, D), lambda i, f: (i, 0)),
        out_shape=jax.ShapeDtypeStruct((T, D), jnp.float32),
        compiler_params=pltpu.CompilerParams(
            dimension_semantics=("parallel", "arbitrary")),
    )(x_bf, sWg_bf, sWu_bf, sWd_bf, sgw_t, moe_out)

    return final.reshape(B, S, D), logits
